# Initial kernel scaffold; baseline (speedup 1.0000x reference)
import jax
import jax.numpy as jnp
from jax.experimental import pallas as pl


def kernel(x, edge_attr, edge_index, active_nid, W_ne, b_ne, W_ee, b_ee, W_msg, W_self, b_n, W_eu, b_e, W1, b1, W2, b2):
    raise NotImplementedError("write your pallas kernel here")


# speedup vs baseline: 1.1202x; 1.1202x over previous
import functools

import jax
import jax.numpy as jnp
from jax import lax
from jax.experimental import pallas as pl
from jax.experimental.pallas import tpu as pltpu
from jax.experimental.pallas import tpu_sc as plsc

N = 10000
E = 320000
D = 128
EDGE_DIM = 16
N_ACTIVE = 4096

NC, NS = 2, 16
NW = NC * NS
C = 128
E_PAD = 327680
EPW = E_PAD // NW
NCH = EPW // C
NAGG = 10112
ZR = NAGG // NS

def _sc_mesh():
    return plsc.VectorSubcoreMesh(
        core_axis_name="c", subcore_axis_name="s",
        num_cores=NC, num_subcores=NS)


def _worker_id():
    return lax.axis_index("s") * NC + lax.axis_index("c")


def _relu_rows(dst_ref, *src_refs, rows):
    def row(r, _):
        for cg in range(D // 16):
            sl = pl.ds(cg * 16, 16)
            acc = src_refs[0][r, sl]
            for s in src_refs[1:]:
                acc = acc + s[r, sl]
            dst_ref[r, sl] = jnp.maximum(acc, 0.0)
        return 0
    lax.fori_loop(0, rows, row, 0)


@functools.cache
def _sc_message_fn():
    @functools.partial(
        pl.kernel,
        out_type=jax.ShapeDtypeStruct((NC, NAGG, D), jnp.float32),
        mesh=_sc_mesh(),
        scratch_types=[
            pltpu.VMEM((C,), jnp.int32),
            pltpu.VMEM((C,), jnp.int32),
            pltpu.VMEM((C, D), jnp.float32),
            pltpu.VMEM((C, D), jnp.float32),
            pltpu.VMEM_SHARED((NAGG, D), jnp.float32),
            pltpu.SemaphoreType.DMA,
        ],
    )
    def _sc_message(hm, em, srcp, dstp, zeros, out, idx_s, idx_d, em_v, g_v,
                    agg, sem):
        c = lax.axis_index("c")
        s = lax.axis_index("s")
        w = _worker_id()
        pltpu.sync_copy(zeros.at[pl.ds(s * ZR, ZR)], agg.at[pl.ds(s * ZR, ZR)])
        plsc.subcore_barrier()

        def chunk(i, _):
            base = w * EPW + i * C
            pltpu.sync_copy(srcp.at[pl.ds(base, C)], idx_s)
            pltpu.sync_copy(dstp.at[pl.ds(base, C)], idx_d)
            pltpu.sync_copy(em.at[pl.ds(base, C)], em_v)
            pltpu.async_copy(hm.at[idx_s], g_v, sem).wait()
            _relu_rows(g_v, g_v, em_v, rows=C)
            pltpu.sync_copy(g_v, agg.at[idx_d], add=True)
            return 0

        lax.fori_loop(0, NCH, chunk, 0)
        plsc.subcore_barrier()
        pltpu.sync_copy(agg.at[pl.ds(s * ZR, ZR)],
                        out.at[c, pl.ds(s * ZR, ZR)])

    return _sc_message


@functools.cache
def _sc_edge_update_fn():
    @functools.partial(
        pl.kernel,
        out_type=jax.ShapeDtypeStruct((E_PAD, D), jnp.float32),
        mesh=_sc_mesh(),
        scratch_types=[
            pltpu.VMEM((C,), jnp.int32),
            pltpu.VMEM((C,), jnp.int32),
            pltpu.VMEM((C, D), jnp.float32),
            pltpu.VMEM((C, D), jnp.float32),
            pltpu.VMEM((C, D), jnp.float32),
            pltpu.SemaphoreType.DMA,
        ],
    )
    def _sc_edge_update(hu1, hu2, eu, srcp, dstp, out, idx_s, idx_d, g1, g2,
                        ev, sem):
        w = _worker_id()

        def chunk(i, _):
            base = w * EPW + i * C
            pltpu.sync_copy(srcp.at[pl.ds(base, C)], idx_s)
            pltpu.sync_copy(dstp.at[pl.ds(base, C)], idx_d)
            pltpu.sync_copy(eu.at[pl.ds(base, C)], ev)
            pltpu.async_copy(hu1.at[idx_s], g1, sem).wait()
            pltpu.async_copy(hu2.at[idx_d], g2, sem).wait()
            _relu_rows(ev, g1, g2, ev, rows=C)
            pltpu.sync_copy(ev, out.at[pl.ds(base, C)])
            return 0

        lax.fori_loop(0, NCH, chunk, 0)

    return _sc_edge_update


_APW = N_ACTIVE // NW


@functools.cache
def _sc_active_gather_fn():
    @functools.partial(
        pl.kernel,
        out_type=jax.ShapeDtypeStruct((N_ACTIVE, D), jnp.float32),
        mesh=_sc_mesh(),
        scratch_types=[
            pltpu.VMEM((_APW,), jnp.int32),
            pltpu.VMEM((_APW, D), jnp.float32),
            pltpu.SemaphoreType.DMA,
        ],
    )
    def _sc_active_gather(h, nid, out, idx_v, rows_v, sem):
        w = _worker_id()
        base = w * _APW
        pltpu.sync_copy(nid.at[pl.ds(base, _APW)], idx_v)
        pltpu.async_copy(h.at[idx_v], rows_v, sem).wait()
        pltpu.sync_copy(rows_v, out.at[pl.ds(base, _APW)])

    return _sc_active_gather


def _dot(a, b):
    return jnp.dot(a, b, preferred_element_type=jnp.float32)


def _encode_nodes_body(x, wne, bne, wmh, wself, bn, hm_o, hs_o):
    h = jnp.maximum(_dot(x[...], wne[...]) + bne[...], 0.0)
    hm_o[...] = _dot(h, wmh[...])
    hs_o[...] = _dot(h, wself[...]) + bn[...]


def _encode_nodes(x, wne, bne, wmh, wself, bn):
    return pl.pallas_call(
        _encode_nodes_body,
        out_shape=[jax.ShapeDtypeStruct((N, D), jnp.float32)] * 2,
    )(x, wne, bne, wmh, wself, bn)


_BE = 4096


def _encode_edges_body(ea, wee, bee, wme, wue, be, em_o, eu_o):
    e = jnp.maximum(_dot(ea[...], wee[...]) + bee[...], 0.0)
    em_o[...] = _dot(e, wme[...])
    eu_o[...] = _dot(e, wue[...]) + be[...]


def _encode_edges(ea, wee, bee, wme, wue, be):
    g = E_PAD // _BE
    blk = lambda r, c: pl.BlockSpec((r, c), lambda i: (0, 0))
    return pl.pallas_call(
        _encode_edges_body,
        grid=(g,),
        in_specs=[pl.BlockSpec((_BE, EDGE_DIM), lambda i: (i, 0)),
                  blk(EDGE_DIM, D), blk(1, D), blk(D, D), blk(D, D), blk(1, D)],
        out_specs=[pl.BlockSpec((_BE, D), lambda i: (i, 0))] * 2,
        out_shape=[jax.ShapeDtypeStruct((E_PAD, D), jnp.float32)] * 2,
    )(ea, wee, bee, wme, wue, be)


def _edge_mm2_body(e, wme, wue, be, em_o, eu_o):
    ev = e[...]
    em_o[...] = _dot(ev, wme[...])
    eu_o[...] = _dot(ev, wue[...]) + be[...]


def _edge_mm2(e, wme, wue, be):
    g = E_PAD // _BE
    blk = lambda r, c: pl.BlockSpec((r, c), lambda i: (0, 0))
    return pl.pallas_call(
        _edge_mm2_body,
        grid=(g,),
        in_specs=[pl.BlockSpec((_BE, D), lambda i: (i, 0)),
                  blk(D, D), blk(D, D), blk(1, D)],
        out_specs=[pl.BlockSpec((_BE, D), lambda i: (i, 0))] * 2,
        out_shape=[jax.ShapeDtypeStruct((E_PAD, D), jnp.float32)] * 2,
    )(e, wme, wue, be)


def _edge_mm1_body(e, wme, em_o):
    em_o[...] = _dot(e[...], wme[...])


def _edge_mm1(e, wme):
    g = E_PAD // _BE
    return pl.pallas_call(
        _edge_mm1_body,
        grid=(g,),
        in_specs=[pl.BlockSpec((_BE, D), lambda i: (i, 0)),
                  pl.BlockSpec((D, D), lambda i: (0, 0))],
        out_specs=pl.BlockSpec((_BE, D), lambda i: (i, 0)),
        out_shape=jax.ShapeDtypeStruct((E_PAD, D), jnp.float32),
    )(e, wme)


def _node_update_mid_body(hs, agg, wmh, wself, bn, wes, wed,
                          hm_o, hs_o, hu1_o, hu2_o):
    a = agg[...]
    h = jnp.maximum(hs[...] + a[0, :N, :] + a[1, :N, :], 0.0)
    hm_o[...] = _dot(h, wmh[...])
    hs_o[...] = _dot(h, wself[...]) + bn[...]
    hu1_o[...] = _dot(h, wes[...])
    hu2_o[...] = _dot(h, wed[...])


def _node_update_mid(hs, agg, wmh, wself, bn, wes, wed):
    return pl.pallas_call(
        _node_update_mid_body,
        out_shape=[jax.ShapeDtypeStruct((N, D), jnp.float32)] * 4,
    )(hs, agg, wmh, wself, bn, wes, wed)


def _node_update_last_body(hs, agg, h_o):
    a = agg[...]
    h_o[...] = jnp.maximum(hs[...] + a[0, :N, :] + a[1, :N, :], 0.0)


def _node_update_last(hs, agg):
    return pl.pallas_call(
        _node_update_last_body,
        out_shape=jax.ShapeDtypeStruct((N, D), jnp.float32),
    )(hs, agg)


def _mlp_body(ha, w1, b1, w2, b2, out):
    z = jnp.maximum(_dot(ha[...], w1[...]) + b1[...], 0.0)
    out[...] = _dot(z, w2[...]) + b2[...]


def _mlp(ha, w1, b1, w2, b2):
    return pl.pallas_call(
        _mlp_body,
        out_shape=jax.ShapeDtypeStruct((N_ACTIVE, 1), jnp.float32),
    )(ha, w1, b1, w2, b2)


def kernel(x, edge_attr, edge_index, active_nid, W_ne, b_ne, W_ee, b_ee,
           W_msg, W_self, b_n, W_eu, b_e, W1, b1, W2, b2):
    f32 = jnp.float32
    src = edge_index[0].astype(jnp.int32)
    dst = edge_index[1].astype(jnp.int32)
    npad = E_PAD - E
    src_p = jnp.concatenate([src, jnp.zeros((npad,), jnp.int32)])
    dst_p = jnp.concatenate([dst, jnp.full((npad,), N, jnp.int32)])
    ea_p = jnp.concatenate([edge_attr, jnp.zeros((npad, EDGE_DIM), f32)])
    zeros_agg = jnp.zeros((NAGG, D), f32)

    Wm_h, Wm_e = W_msg[:D], W_msg[D:]
    We_s, We_d, We_e = W_eu[:D], W_eu[D:2 * D], W_eu[2 * D:]
    row = lambda b: b.reshape(1, -1)

    hm, hs = _encode_nodes(x, W_ne, row(b_ne), Wm_h, W_self, row(b_n))
    em, eu = _encode_edges(ea_p, W_ee, row(b_ee), Wm_e, We_e, row(b_e))

    for layer in range(3):
        agg = _sc_message_fn()(hm, em, src_p, dst_p, zeros_agg)
        if layer < 2:
            hm, hs, hu1, hu2 = _node_update_mid(
                hs, agg, Wm_h, W_self, row(b_n), We_s, We_d)
            e = _sc_edge_update_fn()(hu1, hu2, eu, src_p, dst_p)
            if layer == 0:
                em, eu = _edge_mm2(e, Wm_e, We_e, row(b_e))
            else:
                em = _edge_mm1(e, Wm_e)
        else:
            h_fin = _node_update_last(hs, agg)

    ha = _sc_active_gather_fn()(h_fin, active_nid)
    logits = _mlp(ha, W1, row(b1), W2, b2.reshape(1, 1))
    return (logits, active_nid)


# speedup vs baseline: 1.7708x; 1.5808x over previous
import functools

import jax
import jax.numpy as jnp
from jax import lax
from jax.experimental import pallas as pl
from jax.experimental.pallas import tpu as pltpu
from jax.experimental.pallas import tpu_sc as plsc

N = 10000
E = 320000
D = 128
EDGE_DIM = 16
N_ACTIVE = 4096

NC, NS = 2, 16
NW = NC * NS
C = 128
E_PAD = 327680
EPW = E_PAD // NW
NCH = EPW // C
NAGG = 10112
ZR = NAGG // NS

def _sc_mesh():
    return plsc.VectorSubcoreMesh(
        core_axis_name="c", subcore_axis_name="s",
        num_cores=NC, num_subcores=NS)


def _worker_id():
    return lax.axis_index("s") * NC + lax.axis_index("c")


def _relu_rows(dst_ref, *src_refs, rows):
    def row(r, _):
        for cg in range(D // 16):
            sl = pl.ds(cg * 16, 16)
            acc = src_refs[0][r, sl]
            for s in src_refs[1:]:
                acc = acc + s[r, sl]
            dst_ref[r, sl] = jnp.maximum(acc, 0.0)
        return 0
    lax.fori_loop(0, rows, row, 0)


CM = 80
NCHM = EPW // CM
SLAB = 16


@functools.cache
def _sc_message_fn():
    @functools.partial(
        pl.kernel,
        out_type=jax.ShapeDtypeStruct((NC, NAGG, D), jnp.float32),
        mesh=_sc_mesh(),
        scratch_types=[
            pltpu.VMEM((SLAB, CM), jnp.int32),
            pltpu.VMEM((SLAB, CM), jnp.int32),
            pltpu.VMEM((2, CM, D), jnp.float32),
            pltpu.VMEM((2, CM, D), jnp.float32),
            pltpu.VMEM_SHARED((NAGG, D), jnp.float32),
            pltpu.SemaphoreType.DMA,
            pltpu.SemaphoreType.DMA,
            pltpu.SemaphoreType.DMA,
            pltpu.SemaphoreType.DMA,
        ],
    )
    def _sc_message(hm, em, srcp, dstp, zeros, out, src_v, dst_v, em_v, g_v,
                    agg, se0, se1, sg0, sg1):
        c = lax.axis_index("c")
        s = lax.axis_index("s")
        w = _worker_id()
        se = (se0, se1)
        sg = (sg0, sg1)
        pltpu.sync_copy(zeros.at[pl.ds(s * ZR, ZR)], agg.at[pl.ds(s * ZR, ZR)])
        plsc.subcore_barrier()

        def load_slab(i):
            i = pl.multiple_of(i, SLAB)
            pltpu.sync_copy(srcp.at[w, pl.ds(i, SLAB)], src_v)
            pltpu.sync_copy(dstp.at[w, pl.ds(i, SLAB)], dst_v)

        def start(i, b):
            pltpu.async_copy(em.at[pl.ds(w * EPW + i * CM, CM)], em_v.at[b],
                             se[b])
            pltpu.async_copy(hm.at[src_v.at[i % SLAB]], g_v.at[b], sg[b])

        load_slab(0)
        start(0, 0)

        def pair(j, _):
            for b in (0, 1):
                i = 2 * j + b
                nxt = i + 1

                @pl.when((nxt < NCHM) & (nxt % SLAB != 0))
                def _():
                    start(nxt, 1 - b)

                pltpu.make_async_copy(em.at[pl.ds(0, CM)], em_v.at[b],
                                      se[b]).wait()
                pltpu.make_async_copy(hm.at[pl.ds(0, CM)], g_v.at[b],
                                      sg[b]).wait()
                _relu_rows(g_v.at[b], g_v.at[b], em_v.at[b], rows=CM)
                pltpu.sync_copy(g_v.at[b], agg.at[dst_v.at[i % SLAB]],
                                add=True)

                @pl.when((nxt < NCHM) & (nxt % SLAB == 0))
                def _():
                    load_slab(nxt)
                    start(nxt, 1 - b)
            return 0

        lax.fori_loop(0, NCHM // 2, pair, 0)
        plsc.subcore_barrier()
        pltpu.sync_copy(agg.at[pl.ds(s * ZR, ZR)],
                        out.at[c, pl.ds(s * ZR, ZR)])

    return _sc_message


@functools.cache
def _sc_edge_update_fn():
    @functools.partial(
        pl.kernel,
        out_type=jax.ShapeDtypeStruct((E_PAD, D), jnp.float32),
        mesh=_sc_mesh(),
        scratch_types=[
            pltpu.VMEM((NCH, C), jnp.int32),
            pltpu.VMEM((NCH, C), jnp.int32),
            pltpu.VMEM((2, C, D), jnp.float32),
            pltpu.VMEM((2, C, D), jnp.float32),
            pltpu.VMEM((2, C, D), jnp.float32),
            pltpu.SemaphoreType.DMA,
            pltpu.SemaphoreType.DMA,
            pltpu.SemaphoreType.DMA,
            pltpu.SemaphoreType.DMA,
            pltpu.SemaphoreType.DMA,
            pltpu.SemaphoreType.DMA,
        ],
    )
    def _sc_edge_update(hu1, hu2, eu, srcp, dstp, out, src_v, dst_v, g1, g2,
                        ev, sv0, sv1, sa0, sa1, sb0, sb1):
        w = _worker_id()
        sv = (sv0, sv1)
        sa = (sa0, sa1)
        sb = (sb0, sb1)
        pltpu.sync_copy(srcp.at[w], src_v)
        pltpu.sync_copy(dstp.at[w], dst_v)

        def start(i, b):
            pltpu.async_copy(eu.at[pl.ds(w * EPW + i * C, C)], ev.at[b],
                             sv[b])
            pltpu.async_copy(hu1.at[src_v.at[i]], g1.at[b], sa[b])
            pltpu.async_copy(hu2.at[dst_v.at[i]], g2.at[b], sb[b])

        start(0, 0)

        def pair(j, _):
            for b in (0, 1):
                i = 2 * j + b
                nxt = i + 1

                @pl.when(nxt < NCH)
                def _():
                    start(nxt, 1 - b)

                pltpu.make_async_copy(eu.at[pl.ds(0, C)], ev.at[b],
                                      sv[b]).wait()
                pltpu.make_async_copy(hu1.at[pl.ds(0, C)], g1.at[b],
                                      sa[b]).wait()
                pltpu.make_async_copy(hu2.at[pl.ds(0, C)], g2.at[b],
                                      sb[b]).wait()
                _relu_rows(ev.at[b], g1.at[b], g2.at[b], ev.at[b], rows=C)
                pltpu.sync_copy(ev.at[b], out.at[pl.ds(w * EPW + i * C, C)])
            return 0

        lax.fori_loop(0, NCH // 2, pair, 0)

    return _sc_edge_update


_APW = N_ACTIVE // NW


@functools.cache
def _sc_active_gather_fn():
    @functools.partial(
        pl.kernel,
        out_type=jax.ShapeDtypeStruct((N_ACTIVE, D), jnp.float32),
        mesh=_sc_mesh(),
        scratch_types=[
            pltpu.VMEM((_APW,), jnp.int32),
            pltpu.VMEM((_APW, D), jnp.float32),
            pltpu.SemaphoreType.DMA,
        ],
    )
    def _sc_active_gather(h, nid, out, idx_v, rows_v, sem):
        w = _worker_id()
        base = w * _APW
        pltpu.sync_copy(nid.at[pl.ds(base, _APW)], idx_v)
        pltpu.async_copy(h.at[idx_v], rows_v, sem).wait()
        pltpu.sync_copy(rows_v, out.at[pl.ds(base, _APW)])

    return _sc_active_gather


def _dot(a, b):
    return jnp.dot(a, b, preferred_element_type=jnp.float32)


def _encode_nodes_body(x, wne, bne, wmh, wself, bn, hm_o, hs_o):
    h = jnp.maximum(_dot(x[...], wne[...]) + bne[...], 0.0)
    hm_o[...] = _dot(h, wmh[...])
    hs_o[...] = _dot(h, wself[...]) + bn[...]


def _encode_nodes(x, wne, bne, wmh, wself, bn):
    return pl.pallas_call(
        _encode_nodes_body,
        out_shape=[jax.ShapeDtypeStruct((N, D), jnp.float32)] * 2,
    )(x, wne, bne, wmh, wself, bn)


_BE = 4096


def _encode_edges_body(ea, wee, bee, wme, wue, be, em_o, eu_o):
    e = jnp.maximum(_dot(ea[...], wee[...]) + bee[...], 0.0)
    em_o[...] = _dot(e, wme[...])
    eu_o[...] = _dot(e, wue[...]) + be[...]


def _encode_edges(ea, wee, bee, wme, wue, be):
    g = E_PAD // _BE
    blk = lambda r, c: pl.BlockSpec((r, c), lambda i: (0, 0))
    return pl.pallas_call(
        _encode_edges_body,
        grid=(g,),
        in_specs=[pl.BlockSpec((_BE, EDGE_DIM), lambda i: (i, 0)),
                  blk(EDGE_DIM, D), blk(1, D), blk(D, D), blk(D, D), blk(1, D)],
        out_specs=[pl.BlockSpec((_BE, D), lambda i: (i, 0))] * 2,
        out_shape=[jax.ShapeDtypeStruct((E_PAD, D), jnp.float32)] * 2,
    )(ea, wee, bee, wme, wue, be)


def _edge_mm2_body(e, wme, wue, be, em_o, eu_o):
    ev = e[...]
    em_o[...] = _dot(ev, wme[...])
    eu_o[...] = _dot(ev, wue[...]) + be[...]


def _edge_mm2(e, wme, wue, be):
    g = E_PAD // _BE
    blk = lambda r, c: pl.BlockSpec((r, c), lambda i: (0, 0))
    return pl.pallas_call(
        _edge_mm2_body,
        grid=(g,),
        in_specs=[pl.BlockSpec((_BE, D), lambda i: (i, 0)),
                  blk(D, D), blk(D, D), blk(1, D)],
        out_specs=[pl.BlockSpec((_BE, D), lambda i: (i, 0))] * 2,
        out_shape=[jax.ShapeDtypeStruct((E_PAD, D), jnp.float32)] * 2,
    )(e, wme, wue, be)


def _edge_mm1_body(e, wme, em_o):
    em_o[...] = _dot(e[...], wme[...])


def _edge_mm1(e, wme):
    g = E_PAD // _BE
    return pl.pallas_call(
        _edge_mm1_body,
        grid=(g,),
        in_specs=[pl.BlockSpec((_BE, D), lambda i: (i, 0)),
                  pl.BlockSpec((D, D), lambda i: (0, 0))],
        out_specs=pl.BlockSpec((_BE, D), lambda i: (i, 0)),
        out_shape=jax.ShapeDtypeStruct((E_PAD, D), jnp.float32),
    )(e, wme)


def _node_update_mid_body(hs, agg, wmh, wself, bn, wes, wed,
                          hm_o, hs_o, hu1_o, hu2_o):
    a = agg[...]
    h = jnp.maximum(hs[...] + a[0, :N, :] + a[1, :N, :], 0.0)
    hm_o[...] = _dot(h, wmh[...])
    hs_o[...] = _dot(h, wself[...]) + bn[...]
    hu1_o[...] = _dot(h, wes[...])
    hu2_o[...] = _dot(h, wed[...])


def _node_update_mid(hs, agg, wmh, wself, bn, wes, wed):
    return pl.pallas_call(
        _node_update_mid_body,
        out_shape=[jax.ShapeDtypeStruct((N, D), jnp.float32)] * 4,
    )(hs, agg, wmh, wself, bn, wes, wed)


def _node_update_last_body(hs, agg, h_o):
    a = agg[...]
    h_o[...] = jnp.maximum(hs[...] + a[0, :N, :] + a[1, :N, :], 0.0)


def _node_update_last(hs, agg):
    return pl.pallas_call(
        _node_update_last_body,
        out_shape=jax.ShapeDtypeStruct((N, D), jnp.float32),
    )(hs, agg)


def _mlp_body(ha, w1, b1, w2, b2, out):
    z = jnp.maximum(_dot(ha[...], w1[...]) + b1[...], 0.0)
    out[...] = _dot(z, w2[...]) + b2[...]


def _mlp(ha, w1, b1, w2, b2):
    return pl.pallas_call(
        _mlp_body,
        out_shape=jax.ShapeDtypeStruct((N_ACTIVE, 1), jnp.float32),
    )(ha, w1, b1, w2, b2)


def kernel(x, edge_attr, edge_index, active_nid, W_ne, b_ne, W_ee, b_ee,
           W_msg, W_self, b_n, W_eu, b_e, W1, b1, W2, b2):
    f32 = jnp.float32
    src = edge_index[0].astype(jnp.int32)
    dst = edge_index[1].astype(jnp.int32)
    npad = E_PAD - E
    src_p = jnp.concatenate([src, jnp.zeros((npad,), jnp.int32)])
    dst_p = jnp.concatenate([dst, jnp.full((npad,), N, jnp.int32)])
    src_pm = src_p.reshape(NW, NCHM, CM)
    dst_pm = dst_p.reshape(NW, NCHM, CM)
    src_pe = src_p.reshape(NW, NCH, C)
    dst_pe = dst_p.reshape(NW, NCH, C)
    ea_p = jnp.concatenate([edge_attr, jnp.zeros((npad, EDGE_DIM), f32)])
    zeros_agg = jnp.zeros((NAGG, D), f32)

    Wm_h, Wm_e = W_msg[:D], W_msg[D:]
    We_s, We_d, We_e = W_eu[:D], W_eu[D:2 * D], W_eu[2 * D:]
    row = lambda b: b.reshape(1, -1)

    hm, hs = _encode_nodes(x, W_ne, row(b_ne), Wm_h, W_self, row(b_n))
    em, eu = _encode_edges(ea_p, W_ee, row(b_ee), Wm_e, We_e, row(b_e))

    for layer in range(3):
        agg = _sc_message_fn()(hm, em, src_pm, dst_pm, zeros_agg)
        if layer < 2:
            hm, hs, hu1, hu2 = _node_update_mid(
                hs, agg, Wm_h, W_self, row(b_n), We_s, We_d)
            e = _sc_edge_update_fn()(hu1, hu2, eu, src_pe, dst_pe)
            if layer == 0:
                em, eu = _edge_mm2(e, Wm_e, We_e, row(b_e))
            else:
                em = _edge_mm1(e, Wm_e)
        else:
            h_fin = _node_update_last(hs, agg)

    ha = _sc_active_gather_fn()(h_fin, active_nid)
    logits = _mlp(ha, W1, row(b1), W2, b2.reshape(1, 1))
    return (logits, active_nid)


# speedup vs baseline: 3.5689x; 2.0154x over previous
import functools

import jax
import jax.numpy as jnp
from jax import lax
from jax.experimental import pallas as pl
from jax.experimental.pallas import tpu as pltpu
from jax.experimental.pallas import tpu_sc as plsc

N = 10000
E = 320000
D = 128
EDGE_DIM = 16
N_ACTIVE = 4096

NC, NS = 2, 16
NW = NC * NS
C = 128
E_PAD = 327680
EPW = E_PAD // NW
NCH = EPW // C
NAGG = 10112
ZR = NAGG // NS

def _sc_mesh():
    return plsc.VectorSubcoreMesh(
        core_axis_name="c", subcore_axis_name="s",
        num_cores=NC, num_subcores=NS)


def _worker_id():
    return lax.axis_index("s") * NC + lax.axis_index("c")


def _relu_rows(dst_ref, *src_refs, rows):
    def row(r, _):
        for cg in range(D // 16):
            sl = pl.ds(cg * 16, 16)
            acc = src_refs[0][r, sl]
            for s in src_refs[1:]:
                acc = acc + s[r, sl]
            dst_ref[r, sl] = jnp.maximum(acc, 0.0)
        return 0
    lax.fori_loop(0, rows, row, 0)


CM = 80
NCHM = EPW // CM
SLAB = 16


@functools.cache
def _sc_message_fn():
    @functools.partial(
        pl.kernel,
        out_type=jax.ShapeDtypeStruct((NC, NAGG, D), jnp.float32),
        mesh=_sc_mesh(),
        scratch_types=[
            pltpu.VMEM((SLAB, CM), jnp.int32),
            pltpu.VMEM((SLAB, CM), jnp.int32),
            pltpu.VMEM((2, CM, D), jnp.float32),
            pltpu.VMEM((2, CM, D), jnp.float32),
            pltpu.VMEM_SHARED((NAGG, D), jnp.float32),
            pltpu.SemaphoreType.DMA,
            pltpu.SemaphoreType.DMA,
            pltpu.SemaphoreType.DMA,
            pltpu.SemaphoreType.DMA,
        ],
    )
    def _sc_message(hm, em, srcp, dstp, zeros, out, src_v, dst_v, em_v, g_v,
                    agg, se0, se1, sg0, sg1):
        c = lax.axis_index("c")
        s = lax.axis_index("s")
        w = _worker_id()
        se = (se0, se1)
        sg = (sg0, sg1)
        pltpu.sync_copy(zeros.at[pl.ds(s * ZR, ZR)], agg.at[pl.ds(s * ZR, ZR)])
        plsc.subcore_barrier()

        def load_slab(i):
            i = pl.multiple_of(i, SLAB)
            pltpu.sync_copy(srcp.at[w, pl.ds(i, SLAB)], src_v)
            pltpu.sync_copy(dstp.at[w, pl.ds(i, SLAB)], dst_v)

        def start(i, b):
            pltpu.async_copy(em.at[pl.ds(w * EPW + i * CM, CM)], em_v.at[b],
                             se[b])
            pltpu.async_copy(hm.at[src_v.at[i % SLAB]], g_v.at[b], sg[b])

        load_slab(0)
        start(0, 0)

        def pair(j, _):
            for b in (0, 1):
                i = 2 * j + b
                nxt = i + 1

                @pl.when((nxt < NCHM) & (nxt % SLAB != 0))
                def _():
                    start(nxt, 1 - b)

                pltpu.make_async_copy(em.at[pl.ds(0, CM)], em_v.at[b],
                                      se[b]).wait()
                pltpu.make_async_copy(hm.at[pl.ds(0, CM)], g_v.at[b],
                                      sg[b]).wait()
                _relu_rows(g_v.at[b], g_v.at[b], em_v.at[b], rows=CM)
                pltpu.sync_copy(g_v.at[b], agg.at[dst_v.at[i % SLAB]],
                                add=True)

                @pl.when((nxt < NCHM) & (nxt % SLAB == 0))
                def _():
                    load_slab(nxt)
                    start(nxt, 1 - b)
            return 0

        lax.fori_loop(0, NCHM // 2, pair, 0)
        plsc.subcore_barrier()
        pltpu.sync_copy(agg.at[pl.ds(s * ZR, ZR)],
                        out.at[c, pl.ds(s * ZR, ZR)])

    return _sc_message


@functools.cache
def _sc_edge_update_fn():
    @functools.partial(
        pl.kernel,
        out_type=jax.ShapeDtypeStruct((E_PAD, D), jnp.float32),
        mesh=_sc_mesh(),
        scratch_types=[
            pltpu.VMEM((NCH, C), jnp.int32),
            pltpu.VMEM((NCH, C), jnp.int32),
            pltpu.VMEM((2, C, D), jnp.float32),
            pltpu.VMEM((2, C, D), jnp.float32),
            pltpu.VMEM((2, C, D), jnp.float32),
            pltpu.SemaphoreType.DMA,
            pltpu.SemaphoreType.DMA,
            pltpu.SemaphoreType.DMA,
            pltpu.SemaphoreType.DMA,
            pltpu.SemaphoreType.DMA,
            pltpu.SemaphoreType.DMA,
        ],
    )
    def _sc_edge_update(hu1, hu2, eu, srcp, dstp, out, src_v, dst_v, g1, g2,
                        ev, sv0, sv1, sa0, sa1, sb0, sb1):
        w = _worker_id()
        sv = (sv0, sv1)
        sa = (sa0, sa1)
        sb = (sb0, sb1)
        pltpu.sync_copy(srcp.at[w], src_v)
        pltpu.sync_copy(dstp.at[w], dst_v)

        def start(i, b):
            pltpu.async_copy(eu.at[pl.ds(w * EPW + i * C, C)], ev.at[b],
                             sv[b])
            pltpu.async_copy(hu1.at[src_v.at[i]], g1.at[b], sa[b])
            pltpu.async_copy(hu2.at[dst_v.at[i]], g2.at[b], sb[b])

        start(0, 0)

        def pair(j, _):
            for b in (0, 1):
                i = 2 * j + b
                nxt = i + 1

                @pl.when(nxt < NCH)
                def _():
                    start(nxt, 1 - b)

                pltpu.make_async_copy(eu.at[pl.ds(0, C)], ev.at[b],
                                      sv[b]).wait()
                pltpu.make_async_copy(hu1.at[pl.ds(0, C)], g1.at[b],
                                      sa[b]).wait()
                pltpu.make_async_copy(hu2.at[pl.ds(0, C)], g2.at[b],
                                      sb[b]).wait()
                _relu_rows(ev.at[b], g1.at[b], g2.at[b], ev.at[b], rows=C)
                pltpu.sync_copy(ev.at[b], out.at[pl.ds(w * EPW + i * C, C)])
            return 0

        lax.fori_loop(0, NCH // 2, pair, 0)

    return _sc_edge_update


_APW = N_ACTIVE // NW


@functools.cache
def _sc_active_gather_fn():
    @functools.partial(
        pl.kernel,
        out_type=jax.ShapeDtypeStruct((N_ACTIVE, D), jnp.float32),
        mesh=_sc_mesh(),
        scratch_types=[
            pltpu.VMEM((_APW,), jnp.int32),
            pltpu.VMEM((_APW, D), jnp.float32),
            pltpu.SemaphoreType.DMA,
        ],
    )
    def _sc_active_gather(h, nid, out, idx_v, rows_v, sem):
        w = _worker_id()
        base = w * _APW
        pltpu.sync_copy(nid.at[pl.ds(base, _APW)], idx_v)
        pltpu.async_copy(h.at[idx_v], rows_v, sem).wait()
        pltpu.sync_copy(rows_v, out.at[pl.ds(base, _APW)])

    return _sc_active_gather


def _dot(a, b):
    return jnp.dot(a, b, preferred_element_type=jnp.float32)


def _encode_nodes_body(x, wne, bne, wmh, wself, bn, hm_o, hs_o):
    h = jnp.maximum(_dot(x[...], wne[...]) + bne[...], 0.0)
    hm_o[...] = _dot(h, wmh[...])
    hs_o[...] = _dot(h, wself[...]) + bn[...]


def _encode_nodes(x, wne, bne, wmh, wself, bn):
    return pl.pallas_call(
        _encode_nodes_body,
        out_shape=[jax.ShapeDtypeStruct((N, D), jnp.float32)] * 2,
    )(x, wne, bne, wmh, wself, bn)


_BE = 4096


def _encode_edges_body(ea, wee, bee, wme, wue, be, em_o, eu_o):
    e = jnp.maximum(_dot(ea[...], wee[...]) + bee[...], 0.0)
    em_o[...] = _dot(e, wme[...])
    eu_o[...] = _dot(e, wue[...]) + be[...]


def _encode_edges(ea, wee, bee, wme, wue, be):
    g = E_PAD // _BE
    blk = lambda r, c: pl.BlockSpec((r, c), lambda i: (0, 0))
    return pl.pallas_call(
        _encode_edges_body,
        grid=(g,),
        in_specs=[pl.BlockSpec((_BE, EDGE_DIM), lambda i: (i, 0)),
                  blk(EDGE_DIM, D), blk(1, D), blk(D, D), blk(D, D), blk(1, D)],
        out_specs=[pl.BlockSpec((_BE, D), lambda i: (i, 0))] * 2,
        out_shape=[jax.ShapeDtypeStruct((E_PAD, D), jnp.float32)] * 2,
    )(ea, wee, bee, wme, wue, be)


def _edge_mm2_body(e, wme, wue, be, em_o, eu_o):
    ev = e[...]
    em_o[...] = _dot(ev, wme[...])
    eu_o[...] = _dot(ev, wue[...]) + be[...]


def _edge_mm2(e, wme, wue, be):
    g = E_PAD // _BE
    blk = lambda r, c: pl.BlockSpec((r, c), lambda i: (0, 0))
    return pl.pallas_call(
        _edge_mm2_body,
        grid=(g,),
        in_specs=[pl.BlockSpec((_BE, D), lambda i: (i, 0)),
                  blk(D, D), blk(D, D), blk(1, D)],
        out_specs=[pl.BlockSpec((_BE, D), lambda i: (i, 0))] * 2,
        out_shape=[jax.ShapeDtypeStruct((E_PAD, D), jnp.float32)] * 2,
    )(e, wme, wue, be)


def _edge_mm1_body(e, wme, em_o):
    em_o[...] = _dot(e[...], wme[...])


def _edge_mm1(e, wme):
    g = E_PAD // _BE
    return pl.pallas_call(
        _edge_mm1_body,
        grid=(g,),
        in_specs=[pl.BlockSpec((_BE, D), lambda i: (i, 0)),
                  pl.BlockSpec((D, D), lambda i: (0, 0))],
        out_specs=pl.BlockSpec((_BE, D), lambda i: (i, 0)),
        out_shape=jax.ShapeDtypeStruct((E_PAD, D), jnp.float32),
    )(e, wme)


def _node_update_mid_body(hs, agg, wmh, wself, bn, wes, wed,
                          hm_o, hs_o, hu1_o, hu2_o):
    a = agg[...]
    h = jnp.maximum(hs[...] + a[0, :N, :] + a[1, :N, :], 0.0)
    hm_o[...] = _dot(h, wmh[...])
    hs_o[...] = _dot(h, wself[...]) + bn[...]
    hu1_o[...] = _dot(h, wes[...])
    hu2_o[...] = _dot(h, wed[...])


def _node_update_mid(hs, agg, wmh, wself, bn, wes, wed):
    return pl.pallas_call(
        _node_update_mid_body,
        out_shape=[jax.ShapeDtypeStruct((N, D), jnp.float32)] * 4,
    )(hs, agg, wmh, wself, bn, wes, wed)


def _node_update_last_body(hs, agg, h_o):
    a = agg[...]
    h_o[...] = jnp.maximum(hs[...] + a[0, :N, :] + a[1, :N, :], 0.0)


def _node_update_last(hs, agg):
    return pl.pallas_call(
        _node_update_last_body,
        out_shape=jax.ShapeDtypeStruct((N, D), jnp.float32),
    )(hs, agg)


def _mlp_body(ha, w1, b1, w2, b2, out):
    z = jnp.maximum(_dot(ha[...], w1[...]) + b1[...], 0.0)
    out[...] = _dot(z, w2[...]) + b2[...]


def _mlp(ha, w1, b1, w2, b2):
    return pl.pallas_call(
        _mlp_body,
        out_shape=jax.ShapeDtypeStruct((N_ACTIVE, 1), jnp.float32),
    )(ha, w1, b1, w2, b2)


def kernel(x, edge_attr, edge_index, active_nid, W_ne, b_ne, W_ee, b_ee,
           W_msg, W_self, b_n, W_eu, b_e, W1, b1, W2, b2):
    f32 = jnp.float32
    src = edge_index[0].astype(jnp.int32)
    dst = edge_index[1].astype(jnp.int32)
    npad = E_PAD - E
    pad_i = jnp.arange(npad, dtype=jnp.int32)
    src_p = jnp.concatenate([src, (pad_i * 97) % N])
    dst_p = jnp.concatenate([dst, N + pad_i % (NAGG - N)])
    src_pm = src_p.reshape(NW, NCHM, CM)
    dst_pm = dst_p.reshape(NW, NCHM, CM)
    src_pe = src_p.reshape(NW, NCH, C)
    dst_pe = dst_p.reshape(NW, NCH, C)
    ea_p = jnp.concatenate([edge_attr, jnp.zeros((npad, EDGE_DIM), f32)])
    zeros_agg = jnp.zeros((NAGG, D), f32)

    Wm_h, Wm_e = W_msg[:D], W_msg[D:]
    We_s, We_d, We_e = W_eu[:D], W_eu[D:2 * D], W_eu[2 * D:]
    row = lambda b: b.reshape(1, -1)

    hm, hs = _encode_nodes(x, W_ne, row(b_ne), Wm_h, W_self, row(b_n))
    em, eu = _encode_edges(ea_p, W_ee, row(b_ee), Wm_e, We_e, row(b_e))

    for layer in range(3):
        agg = _sc_message_fn()(hm, em, src_pm, dst_pm, zeros_agg)
        if layer < 2:
            hm, hs, hu1, hu2 = _node_update_mid(
                hs, agg, Wm_h, W_self, row(b_n), We_s, We_d)
            e = _sc_edge_update_fn()(hu1, hu2, eu, src_pe, dst_pe)
            if layer == 0:
                em, eu = _edge_mm2(e, Wm_e, We_e, row(b_e))
            else:
                em = _edge_mm1(e, Wm_e)
        else:
            h_fin = _node_update_last(hs, agg)

    ha = _sc_active_gather_fn()(h_fin, active_nid)
    logits = _mlp(ha, W1, row(b1), W2, b2.reshape(1, 1))
    return (logits, active_nid)
